# baseline (speedup 1.0000x reference)
import jax
import jax.numpy as jnp
from jax.experimental import pallas as pl
from jax.experimental.pallas import tpu as pltpu

_RBF_START = 0.0
_RBF_END = 8.0
_RBF_STEPS = 16
_ENERGY_SCALE = 0.01
_EPS = 1e-10

_L = 1024
_R = 4096
_BR = 2048

_LOG2E = 1.4426950408889634


def _ff_body(lf_ref, rf_ref, lc_ref, rc_ref, out_ref):
    j = pl.program_id(0)

    d2 = jnp.zeros((_L, _BR), jnp.float32)
    for axis in range(3):
        diff = lc_ref[:, axis][:, None] - rc_ref[axis, :][None, :]
        d2 = d2 + (diff * diff + _EPS)

    c = 2.0 * (_LOG2E ** 0.5)
    dc = jnp.sqrt(d2 * (c * c))

    s = jnp.float32(0.0)
    for e in range(_RBF_STEPS):
        mu_e = _RBF_START + e * (_RBF_END - _RBF_START) / (_RBF_STEPS - 1)
        w = dc - (mu_e * c)
        rbf = jnp.exp2(-(w * w))
        atn = jax.lax.dot_general(
            lf_ref[:, 16 * e:16 * (e + 1)], rf_ref[:, 16 * e:16 * (e + 1)],
            (((1,), (1,)), ((), ())),
            preferred_element_type=jnp.float32)
        s = s + jnp.sum(rbf * atn)
    s = s * _ENERGY_SCALE

    @pl.when(j == 0)
    def _init():
        out_ref[...] = jnp.zeros((1, 128), jnp.float32)

    out_ref[...] += jnp.full((1, 128), s, jnp.float32)


def kernel(lig_feat, rec_feat, lig_coord, rec_coord, weight, bias):
    rc_t = jnp.transpose(rec_coord, (1, 0))
    lf2 = jnp.reshape(lig_feat, (_L, _RBF_STEPS * 16))
    rf2 = jnp.reshape(rec_feat, (_R, _RBF_STEPS * 16))

    grid = (_R // _BR,)
    total = pl.pallas_call(
        _ff_body,
        grid=grid,
        in_specs=[
            pl.BlockSpec((_L, _RBF_STEPS * 16), lambda j: (0, 0)),
            pl.BlockSpec((_BR, _RBF_STEPS * 16), lambda j: (j, 0)),
            pl.BlockSpec((_L, 3), lambda j: (0, 0)),
            pl.BlockSpec((3, _BR), lambda j: (0, j)),
        ],
        out_specs=pl.BlockSpec((1, 128), lambda j: (0, 0)),
        out_shape=jax.ShapeDtypeStruct((1, 128), jnp.float32),
        compiler_params=pltpu.CompilerParams(
            dimension_semantics=("arbitrary",),
        ),
    )(lf2, rf2, lig_coord, rc_t)

    return bias.reshape(()) + total[0, 0] * weight.reshape(())


# speedup vs baseline: 1.2511x; 1.2511x over previous
import jax
import jax.numpy as jnp
from jax.experimental import pallas as pl
from jax.experimental.pallas import tpu as pltpu

_RBF_START = 0.0
_RBF_END = 8.0
_RBF_STEPS = 16
_ENERGY_SCALE = 0.01
_EPS = 1e-10

_L = 1024
_R = 4096
_BR = 512

_LOG2E = 1.4426950408889634


def _ff_body(lf_ref, rf_ref, lc_ref, rc_ref, out_ref):
    j = pl.program_id(0)

    d2 = jnp.zeros((_L, _BR), jnp.float32)
    for axis in range(3):
        diff = lc_ref[:, axis][:, None] - rc_ref[axis, :][None, :]
        d2 = d2 + (diff * diff + _EPS)

    c = 2.0 * (_LOG2E ** 0.5)
    dc = jnp.sqrt(d2 * (c * c))

    s = jnp.float32(0.0)
    for e in range(_RBF_STEPS):
        mu_e = _RBF_START + e * (_RBF_END - _RBF_START) / (_RBF_STEPS - 1)
        w = dc - (mu_e * c)
        rbf = jnp.exp2(-(w * w))
        atn = jax.lax.dot_general(
            lf_ref[:, 16 * e:16 * (e + 1)], rf_ref[:, 16 * e:16 * (e + 1)],
            (((1,), (1,)), ((), ())),
            preferred_element_type=jnp.float32)
        s = s + jnp.sum(rbf * atn)
    s = s * _ENERGY_SCALE

    @pl.when(j == 0)
    def _init():
        out_ref[...] = jnp.zeros((1, 128), jnp.float32)

    out_ref[...] += jnp.full((1, 128), s, jnp.float32)


def kernel(lig_feat, rec_feat, lig_coord, rec_coord, weight, bias):
    rc_t = jnp.transpose(rec_coord, (1, 0))
    lf2 = jnp.reshape(lig_feat, (_L, _RBF_STEPS * 16))
    rf2 = jnp.reshape(rec_feat, (_R, _RBF_STEPS * 16))

    grid = (_R // _BR,)
    total = pl.pallas_call(
        _ff_body,
        grid=grid,
        in_specs=[
            pl.BlockSpec((_L, _RBF_STEPS * 16), lambda j: (0, 0)),
            pl.BlockSpec((_BR, _RBF_STEPS * 16), lambda j: (j, 0)),
            pl.BlockSpec((_L, 3), lambda j: (0, 0)),
            pl.BlockSpec((3, _BR), lambda j: (0, j)),
        ],
        out_specs=pl.BlockSpec((1, 128), lambda j: (0, 0)),
        out_shape=jax.ShapeDtypeStruct((1, 128), jnp.float32),
        compiler_params=pltpu.CompilerParams(
            dimension_semantics=("arbitrary",),
        ),
    )(lf2, rf2, lig_coord, rc_t)

    return bias.reshape(()) + total[0, 0] * weight.reshape(())


# speedup vs baseline: 1.3103x; 1.0473x over previous
import jax
import jax.numpy as jnp
from jax.experimental import pallas as pl
from jax.experimental.pallas import tpu as pltpu

_RBF_START = 0.0
_RBF_END = 8.0
_RBF_STEPS = 16
_ENERGY_SCALE = 0.01
_EPS = 1e-10

_L = 1024
_R = 4096
_BR = 1024

_LOG2E = 1.4426950408889634


def _ff_body(lf_ref, rf_ref, lc_ref, rc_ref, out_ref):
    j = pl.program_id(0)

    d2 = jnp.zeros((_L, _BR), jnp.float32)
    for axis in range(3):
        diff = lc_ref[:, axis][:, None] - rc_ref[axis, :][None, :]
        d2 = d2 + (diff * diff + _EPS)

    c = 2.0 * (_LOG2E ** 0.5)
    q = d2 * (-(c * c))
    dc = jnp.sqrt(-q)

    s = jnp.float32(0.0)
    for e in range(_RBF_STEPS):
        mu_e = _RBF_START + e * (_RBF_END - _RBF_START) / (_RBF_STEPS - 1)
        rbf = jnp.exp2(dc * (2.0 * mu_e * c) + (q - mu_e * mu_e * c * c))
        atn = jax.lax.dot_general(
            lf_ref[:, 16 * e:16 * (e + 1)], rf_ref[:, 16 * e:16 * (e + 1)],
            (((1,), (1,)), ((), ())),
            preferred_element_type=jnp.float32)
        s = s + jnp.sum(rbf * atn)
    s = s * _ENERGY_SCALE

    @pl.when(j == 0)
    def _init():
        out_ref[...] = jnp.zeros((1, 128), jnp.float32)

    out_ref[...] += jnp.full((1, 128), s, jnp.float32)


def kernel(lig_feat, rec_feat, lig_coord, rec_coord, weight, bias):
    rc_t = jnp.transpose(rec_coord, (1, 0))
    lf2 = jnp.reshape(lig_feat, (_L, _RBF_STEPS * 16))
    rf2 = jnp.reshape(rec_feat, (_R, _RBF_STEPS * 16))

    grid = (_R // _BR,)
    total = pl.pallas_call(
        _ff_body,
        grid=grid,
        in_specs=[
            pl.BlockSpec((_L, _RBF_STEPS * 16), lambda j: (0, 0)),
            pl.BlockSpec((_BR, _RBF_STEPS * 16), lambda j: (j, 0)),
            pl.BlockSpec((_L, 3), lambda j: (0, 0)),
            pl.BlockSpec((3, _BR), lambda j: (0, j)),
        ],
        out_specs=pl.BlockSpec((1, 128), lambda j: (0, 0)),
        out_shape=jax.ShapeDtypeStruct((1, 128), jnp.float32),
        compiler_params=pltpu.CompilerParams(
            dimension_semantics=("arbitrary",),
        ),
    )(lf2, rf2, lig_coord, rc_t)

    return bias.reshape(()) + total[0, 0] * weight.reshape(())


# speedup vs baseline: 1.4593x; 1.1137x over previous
import jax
import jax.numpy as jnp
from jax.experimental import pallas as pl
from jax.experimental.pallas import tpu as pltpu

_RBF_START = 0.0
_RBF_END = 8.0
_RBF_STEPS = 16
_ENERGY_SCALE = 0.01
_EPS = 1e-10

_L = 1024
_R = 4096
_BR = 1024

_LOG2E = 1.4426950408889634


def _ff_body(lf_ref, rf_ref, lc_ref, rc_ref, out_ref):
    j = pl.program_id(0)

    ones_row = jnp.ones((1, _BR), jnp.float32)
    d2 = jnp.zeros((_L, _BR), jnp.float32)
    for axis in range(3):
        lcb = jax.lax.dot_general(
            lc_ref[:, axis][:, None], ones_row,
            (((1,), (0,)), ((), ())),
            preferred_element_type=jnp.float32)
        diff = lcb - rc_ref[axis, :][None, :]
        d2 = d2 + (diff * diff + _EPS)

    c = 2.0 * (_LOG2E ** 0.5)
    dc = jnp.sqrt(d2 * (c * c))

    acc = jnp.zeros((_L, _BR), jnp.float32)
    for e in range(_RBF_STEPS):
        mu_e = _RBF_START + e * (_RBF_END - _RBF_START) / (_RBF_STEPS - 1)
        w = dc - (mu_e * c)
        rbf = jnp.exp2(-(w * w))
        atn = jax.lax.dot_general(
            lf_ref[:, 16 * e:16 * (e + 1)], rf_ref[:, 16 * e:16 * (e + 1)],
            (((1,), (1,)), ((), ())),
            preferred_element_type=jnp.float32)
        acc = acc + rbf * atn
    s = jnp.sum(acc) * _ENERGY_SCALE

    @pl.when(j == 0)
    def _init():
        out_ref[...] = jnp.zeros((1, 128), jnp.float32)

    out_ref[...] += jnp.full((1, 128), s, jnp.float32)


def kernel(lig_feat, rec_feat, lig_coord, rec_coord, weight, bias):
    rc_t = jnp.transpose(rec_coord, (1, 0))
    lf2 = jnp.reshape(lig_feat, (_L, _RBF_STEPS * 16))
    rf2 = jnp.reshape(rec_feat, (_R, _RBF_STEPS * 16))

    grid = (_R // _BR,)
    total = pl.pallas_call(
        _ff_body,
        grid=grid,
        in_specs=[
            pl.BlockSpec((_L, _RBF_STEPS * 16), lambda j: (0, 0)),
            pl.BlockSpec((_BR, _RBF_STEPS * 16), lambda j: (j, 0)),
            pl.BlockSpec((_L, 3), lambda j: (0, 0)),
            pl.BlockSpec((3, _BR), lambda j: (0, j)),
        ],
        out_specs=pl.BlockSpec((1, 128), lambda j: (0, 0)),
        out_shape=jax.ShapeDtypeStruct((1, 128), jnp.float32),
        compiler_params=pltpu.CompilerParams(
            dimension_semantics=("arbitrary",),
        ),
    )(lf2, rf2, lig_coord, rc_t)

    return bias.reshape(()) + total[0, 0] * weight.reshape(())
